# baseline (speedup 1.0000x reference)
import functools

import jax
import jax.numpy as jnp
from jax import lax
from jax.experimental import pallas as pl
from jax.experimental.pallas import tpu as pltpu
from jax.experimental.pallas import tpu_sc as plsc

N_TOK = 8192
N_EXP = 64
K = 8
BLOCK = 1024

NUM_WORKERS = 32
RPW = N_TOK // NUM_WORKERS
L = 16
NG = N_EXP // L


def _probs_kernel(x_ref, w_ref, p_ref):
    x = x_ref[...]
    w = w_ref[...]
    logits = jax.lax.dot_general(
        x, w, (((1,), (1,)), ((), ())), preferred_element_type=jnp.float32
    )
    m = jnp.max(logits, axis=1, keepdims=True)
    e = jnp.exp(logits - m)
    s = jnp.sum(e, axis=1, keepdims=True)
    p_ref[...] = e / s


def _tc_probs(hidden_states, weight):
    return pl.pallas_call(
        _probs_kernel,
        grid=(N_TOK // BLOCK,),
        in_specs=[
            pl.BlockSpec((BLOCK, N_EXP), lambda i: (i, 0)),
            pl.BlockSpec((N_EXP, N_EXP), lambda i: (0, 0)),
        ],
        out_specs=pl.BlockSpec((BLOCK, N_EXP), lambda i: (i, 0)),
        out_shape=jax.ShapeDtypeStruct((N_TOK, N_EXP), jnp.float32),
    )(hidden_states, weight)


@functools.partial(
    pl.kernel,
    out_type=[
        jax.ShapeDtypeStruct((N_TOK // 2, 2 * K), jnp.float32),
        jax.ShapeDtypeStruct((N_TOK // 2, 2 * K), jnp.int32),
    ],
    mesh=plsc.VectorSubcoreMesh(core_axis_name="c", subcore_axis_name="s"),
    compiler_params=pltpu.CompilerParams(needs_layout_passes=False),
    scratch_types=[
        pltpu.VMEM((RPW, N_EXP), jnp.float32),
        pltpu.VMEM((RPW // 2, 2 * K), jnp.float32),
        pltpu.VMEM((RPW // 2, 2 * K), jnp.int32),
    ],
)
def _topk_sc(p_hbm, out_w_hbm, out_i_hbm, buf, ow, oi):
    wid = lax.axis_index("s") * 2 + lax.axis_index("c")
    base = pl.multiple_of(wid * RPW, RPW)
    pltpu.sync_copy(p_hbm.at[pl.ds(base, RPW)], buf)

    lane = lax.iota(jnp.int32, L)
    iotas = [lane + g * L for g in range(NG)]

    def topk_one_row(r, lane_base):
        vs = [buf[r, pl.ds(g * L, L)] for g in range(NG)]
        acc_w = jnp.zeros((L,), jnp.float32)
        acc_i = jnp.zeros((L,), jnp.int32)
        for k in range(K):
            m01 = jnp.maximum(vs[0], vs[1])
            m23 = jnp.maximum(vs[2], vs[3])
            cur = jnp.max(jnp.maximum(m01, m23), axis=0)
            cands = [
                jnp.where(vs[g] == cur, iotas[g], N_EXP) for g in range(NG)
            ]
            c01 = jnp.minimum(cands[0], cands[1])
            c23 = jnp.minimum(cands[2], cands[3])
            idx = jnp.min(jnp.minimum(c01, c23), axis=0)
            ksel = lane == (lane_base + k)
            acc_w = jnp.where(ksel, cur, acc_w)
            acc_i = jnp.where(ksel, idx, acc_i)
            if k + 1 < K:
                vs = [
                    jnp.where(iotas[g] == idx, -1.0, vs[g]) for g in range(NG)
                ]
        return acc_w, acc_i

    def pair_body(p, carry):
        w_a, i_a = topk_one_row(p * 2, 0)
        w_b, i_b = topk_one_row(p * 2 + 1, K)
        ow[p, :] = w_a + w_b
        oi[p, :] = i_a + i_b
        return carry

    lax.fori_loop(0, RPW // 2, pair_body, 0)

    obase = pl.multiple_of(wid * (RPW // 2), RPW // 2)
    pltpu.sync_copy(ow, out_w_hbm.at[pl.ds(obase, RPW // 2)])
    pltpu.sync_copy(oi, out_i_hbm.at[pl.ds(obase, RPW // 2)])


@jax.jit
def kernel(hidden_states, weight):
    probs = _tc_probs(hidden_states, weight)
    out_w2, out_i2 = _topk_sc(probs)
    return (out_w2.reshape(N_TOK, K), out_i2.reshape(N_TOK, K))


# speedup vs baseline: 1.4751x; 1.4751x over previous
import functools

import jax
import jax.numpy as jnp
from jax import lax
from jax.experimental import pallas as pl
from jax.experimental.pallas import tpu as pltpu
from jax.experimental.pallas import tpu_sc as plsc

N_TOK = 8192
N_EXP = 64
K = 8
BLOCK = 1024

NUM_WORKERS = 32
RPW = N_TOK // NUM_WORKERS
L = 16
NG = N_EXP // L


def _probs_kernel(x_ref, w_ref, p_ref):
    x = x_ref[...]
    w = w_ref[...]
    logits = jax.lax.dot_general(
        x, w, (((1,), (1,)), ((), ())), preferred_element_type=jnp.float32
    )
    m = jnp.max(logits, axis=1, keepdims=True)
    e = jnp.exp(logits - m)
    s = jnp.sum(e, axis=1, keepdims=True)
    p_ref[...] = e / s


def _tc_probs(hidden_states, weight):
    return pl.pallas_call(
        _probs_kernel,
        grid=(N_TOK // BLOCK,),
        in_specs=[
            pl.BlockSpec((BLOCK, N_EXP), lambda i: (i, 0)),
            pl.BlockSpec((N_EXP, N_EXP), lambda i: (0, 0)),
        ],
        out_specs=pl.BlockSpec((BLOCK, N_EXP), lambda i: (i, 0)),
        out_shape=jax.ShapeDtypeStruct((N_TOK, N_EXP), jnp.float32),
    )(hidden_states, weight)


@functools.partial(
    pl.kernel,
    out_type=[
        jax.ShapeDtypeStruct((N_TOK // 2, 2 * K), jnp.float32),
        jax.ShapeDtypeStruct((N_TOK // 2, 2 * K), jnp.int32),
    ],
    mesh=plsc.VectorSubcoreMesh(core_axis_name="c", subcore_axis_name="s"),
    compiler_params=pltpu.CompilerParams(needs_layout_passes=False),
    scratch_types=[
        pltpu.VMEM((RPW, N_EXP), jnp.float32),
        pltpu.VMEM((RPW // 2, 2 * K), jnp.float32),
        pltpu.VMEM((RPW // 2, 2 * K), jnp.int32),
    ],
)
def _topk_sc(p_hbm, out_w_hbm, out_i_hbm, buf, ow, oi):
    wid = lax.axis_index("s") * 2 + lax.axis_index("c")
    base = pl.multiple_of(wid * RPW, RPW)
    pltpu.sync_copy(p_hbm.at[pl.ds(base, RPW)], buf)

    lane = lax.iota(jnp.int32, L)
    iotas = [lane + g * L for g in range(NG)]
    lane_next = jnp.minimum(lane + 1, L - 1)
    lane_prev = jnp.maximum(lane - 1, 0)
    lane_m8 = jnp.maximum(lane - K, 0)
    is_last = lane == (L - 1)
    is_first = lane == 0
    lo_half = lane < K

    def _gath(x, i):
        return x.at[i].get(mode="promise_in_bounds")

    def _merge(ak, ai, bk, bi):
        rbk = jnp.flip(bk, 0)
        rbi = jnp.flip(bi, 0)
        take = ak >= rbk
        mk = jnp.where(take, ak, rbk)
        mi = jnp.where(take, ai, rbi)
        return plsc.sort_key_val(mk, mi, descending=True)

    def topk_one_row(r):
        sk, si = [], []
        for g in range(NG):
            k_g, i_g = plsc.sort_key_val(
                buf[r, pl.ds(g * L, L)], iotas[g], descending=True
            )
            sk.append(k_g)
            si.append(i_g)
        k01, i01 = _merge(sk[0], si[0], sk[1], si[1])
        k23, i23 = _merge(sk[2], si[2], sk[3], si[3])
        kf, idxf = _merge(k01, i01, k23, i23)
        kn = jnp.where(is_last, -1.0, _gath(kf, lane_next))
        inx = _gath(idxf, lane_next)
        kp = jnp.where(is_first, -2.0, _gath(kf, lane_prev))
        ipv = _gath(idxf, lane_prev)
        fixed = jnp.where(kf == kn, jnp.minimum(idxf, inx), idxf)
        fixed = jnp.where(kf == kp, jnp.maximum(fixed, ipv), fixed)
        return kf, fixed

    def pair_body(p, carry):
        w_a, i_a = topk_one_row(p * 2)
        w_b, i_b = topk_one_row(p * 2 + 1)
        ow[p, :] = jnp.where(lo_half, w_a, _gath(w_b, lane_m8))
        oi[p, :] = jnp.where(lo_half, i_a, _gath(i_b, lane_m8))
        return carry

    lax.fori_loop(0, RPW // 2, pair_body, 0)

    obase = pl.multiple_of(wid * (RPW // 2), RPW // 2)
    pltpu.sync_copy(ow, out_w_hbm.at[pl.ds(obase, RPW // 2)])
    pltpu.sync_copy(oi, out_i_hbm.at[pl.ds(obase, RPW // 2)])


@jax.jit
def kernel(hidden_states, weight):
    probs = _tc_probs(hidden_states, weight)
    out_w2, out_i2 = _topk_sc(probs)
    return (out_w2.reshape(N_TOK, K), out_i2.reshape(N_TOK, K))


# speedup vs baseline: 1.7029x; 1.1544x over previous
import functools

import jax
import jax.numpy as jnp
from jax import lax
from jax.experimental import pallas as pl
from jax.experimental.pallas import tpu as pltpu
from jax.experimental.pallas import tpu_sc as plsc

N_TOK = 8192
N_EXP = 64
K = 8
BLOCK = 1024

NUM_WORKERS = 32
RPW = N_TOK // NUM_WORKERS
L = 16
NG = N_EXP // L


def _probs_kernel(x_ref, w_ref, p_ref):
    x = x_ref[...]
    w = w_ref[...]
    logits = jax.lax.dot_general(
        x, w, (((1,), (1,)), ((), ())), preferred_element_type=jnp.float32
    )
    m = jnp.max(logits, axis=1, keepdims=True)
    e = jnp.exp(logits - m)
    s = jnp.sum(e, axis=1, keepdims=True)
    p_ref[...] = e / s


def _tc_probs(hidden_states, weight):
    return pl.pallas_call(
        _probs_kernel,
        grid=(N_TOK // BLOCK,),
        in_specs=[
            pl.BlockSpec((BLOCK, N_EXP), lambda i: (i, 0)),
            pl.BlockSpec((N_EXP, N_EXP), lambda i: (0, 0)),
        ],
        out_specs=pl.BlockSpec((BLOCK, N_EXP), lambda i: (i, 0)),
        out_shape=jax.ShapeDtypeStruct((N_TOK, N_EXP), jnp.float32),
    )(hidden_states, weight)


@functools.partial(
    pl.kernel,
    out_type=[
        jax.ShapeDtypeStruct((N_TOK, L), jnp.float32),
        jax.ShapeDtypeStruct((N_TOK, L), jnp.int32),
    ],
    mesh=plsc.VectorSubcoreMesh(core_axis_name="c", subcore_axis_name="s"),
    compiler_params=pltpu.CompilerParams(needs_layout_passes=False),
    scratch_types=[
        pltpu.VMEM((RPW, N_EXP), jnp.float32),
        pltpu.VMEM((RPW, L), jnp.float32),
        pltpu.VMEM((RPW, L), jnp.int32),
    ],
)
def _topk_sc(p_hbm, out_w_hbm, out_i_hbm, buf, ow, oi):
    wid = lax.axis_index("s") * 2 + lax.axis_index("c")
    base = pl.multiple_of(wid * RPW, RPW)
    pltpu.sync_copy(p_hbm.at[pl.ds(base, RPW)], buf)

    lane = lax.iota(jnp.int32, L)
    iotas = [lane + g * L for g in range(NG)]
    lane_next = jnp.minimum(lane + 1, L - 1)
    lane_prev = jnp.maximum(lane - 1, 0)
    lane_m8 = jnp.maximum(lane - K, 0)
    is_last = lane == (L - 1)
    is_first = lane == 0
    lo_half = lane < K

    def _gath(x, i):
        return x.at[i].get(mode="promise_in_bounds")

    def _merge(ak, ai, bk, bi):
        rbk = jnp.flip(bk, 0)
        rbi = jnp.flip(bi, 0)
        take = ak >= rbk
        mk = jnp.where(take, ak, rbk)
        mi = jnp.where(take, ai, rbi)
        return plsc.sort_key_val(mk, mi, descending=True)

    def topk_one_row(r):
        sk, si = [], []
        for g in range(NG):
            k_g, i_g = plsc.sort_key_val(
                buf[r, pl.ds(g * L, L)], iotas[g], descending=True
            )
            sk.append(k_g)
            si.append(i_g)
        k01, i01 = _merge(sk[0], si[0], sk[1], si[1])
        k23, i23 = _merge(sk[2], si[2], sk[3], si[3])
        kf, idxf = _merge(k01, i01, k23, i23)
        kn = jnp.where(is_last, -1.0, _gath(kf, lane_next))
        inx = _gath(idxf, lane_next)
        kp = jnp.where(is_first, -2.0, _gath(kf, lane_prev))
        ipv = _gath(idxf, lane_prev)
        fixed = jnp.where(kf == kn, jnp.minimum(idxf, inx), idxf)
        fixed = jnp.where(kf == kp, jnp.maximum(fixed, ipv), fixed)
        return kf, fixed

    def pair_body(p, carry):
        w_a, i_a = topk_one_row(p * 2)
        w_b, i_b = topk_one_row(p * 2 + 1)
        ow[p * 2, :] = w_a
        oi[p * 2, :] = i_a
        ow[p * 2 + 1, :] = w_b
        oi[p * 2 + 1, :] = i_b
        return carry

    lax.fori_loop(0, RPW // 2, pair_body, 0)

    pltpu.sync_copy(ow, out_w_hbm.at[pl.ds(base, RPW)])
    pltpu.sync_copy(oi, out_i_hbm.at[pl.ds(base, RPW)])


@jax.jit
def kernel(hidden_states, weight):
    probs = _tc_probs(hidden_states, weight)
    out_w, out_i = _topk_sc(probs)
    return (out_w[:, :K], out_i[:, :K])
